# baseline (speedup 1.0000x reference)
import functools
import numpy as np
import jax
import jax.numpy as jnp
from jax import lax
from jax.experimental import pallas as pl
from jax.experimental.pallas import tpu as pltpu
from jax.experimental.pallas import tpu_sc as plsc

NH, NL, NP_, HD = 8, 4, 4, 32
LVL_HW = [(64, 64), (32, 32), (16, 16), (8, 8)]
LVL_SIZES = [h * w for h, w in LVL_HW]
PIX = sum(LVL_SIZES)
LVL_BASE = np.concatenate([[0], np.cumsum(LVL_SIZES)[:-1]])

_l_of_col = (np.arange(128) % 16) // 4
_h_of_col = np.arange(128) // 16
COL_W = np.array([LVL_HW[l][1] for l in _l_of_col], np.float32).reshape(1, 128)
COL_H = np.array([LVL_HW[l][0] for l in _l_of_col], np.float32).reshape(1, 128)
COL_WI = (COL_W.astype(np.int64) * NH).astype(np.int32)
COL_BASE = (np.array([LVL_BASE[l] for l in _l_of_col], np.int64) * NH
            + _h_of_col).astype(np.int32).reshape(1, 128)
PERM_X = np.array([h * 32 + l * 8 + p * 2
                   for h in range(8) for l in range(4) for p in range(4)])
PERM_Y = PERM_X + 1
GSUM = np.kron(np.eye(8, dtype=np.float32), np.ones((16, 16), np.float32))
PERM_LO = np.array([h * 32 + k for h in range(8) for k in range(16)])
PERM_HI = PERM_LO + 16

BP = 544
BQ = 512
CH = 32
CS = CH * 16
NC, NS = 2, 16
NW = NC * NS


def _valproj_body(x_ref, wlo_ref, whi_ref, blo_ref, bhi_ref, o_ref):
    x = x_ref[0]
    ylo = jnp.dot(x, wlo_ref[...], preferred_element_type=jnp.float32) + blo_ref[...]
    yhi = jnp.dot(x, whi_ref[...], preferred_element_type=jnp.float32) + bhi_ref[...]
    lo = lax.bitcast_convert_type(ylo.astype(jnp.bfloat16), jnp.uint16).astype(jnp.uint32)
    hi = lax.bitcast_convert_type(yhi.astype(jnp.bfloat16), jnp.uint16).astype(jnp.uint32)
    o_ref[...] = lo | (hi << 16)


def _samp_body(q_ref, rpx_ref, rpy_ref, wox_ref, woy_ref, box_ref, boy_ref,
               wat_ref, bat_ref, g_ref, colw_ref, colh_ref, colwi_ref,
               colb_ref,
               i0_ref, i1_ref, i2_ref, i3_ref, w0_ref, w1_ref, w2_ref, w3_ref):
    q = q_ref[0]
    sox = jnp.dot(q, wox_ref[...], preferred_element_type=jnp.float32) + box_ref[...]
    soy = jnp.dot(q, woy_ref[...], preferred_element_type=jnp.float32) + boy_ref[...]
    aw = jnp.dot(q, wat_ref[...], preferred_element_type=jnp.float32) + bat_ref[...]
    m = jnp.max(aw, axis=-1, keepdims=True)
    e = jnp.exp(aw - m)
    s = jnp.dot(e, g_ref[...], preferred_element_type=jnp.float32)
    awf = e / s
    locx = rpx_ref[0] + sox
    locy = rpy_ref[0] + soy
    wv = colw_ref[...]
    hv = colh_ref[...]
    ix = locx * wv - 0.5
    iy = locy * hv - 0.5
    ix0 = jnp.floor(ix)
    iy0 = jnp.floor(iy)
    wx1 = ix - ix0
    wx0 = 1.0 - wx1
    wy1 = iy - iy0
    wy0 = 1.0 - wy1
    n = pl.program_id(0)
    nbase = n * (NH * PIX)
    wvi = colwi_ref[...]
    base = colb_ref[...]
    irefs = [i0_ref, i1_ref, i2_ref, i3_ref]
    wrefs = [w0_ref, w1_ref, w2_ref, w3_ref]
    for c, (dy, dx) in enumerate([(0, 0), (0, 1), (1, 0), (1, 1)]):
        fx = ix0 + dx
        fy = iy0 + dy
        valid = (fx >= 0) & (fx <= wv - 1) & (fy >= 0) & (fy <= hv - 1)
        ixc = jnp.clip(fx, 0.0, wv - 1).astype(jnp.int32)
        iyc = jnp.clip(fy, 0.0, hv - 1).astype(jnp.int32)
        row = nbase + base + iyc * wvi + ixc * NH
        wgt = jnp.where(valid, (wx1 if dx else wx0) * (wy1 if dy else wy0), 0.0) * awf
        irefs[c][0] = row
        wrefs[c][0] = wgt


def _out_body(x_ref, w_ref, b_ref, o_ref):
    o_ref[...] = jnp.dot(x_ref[...], w_ref[...],
                         preferred_element_type=jnp.float32) + b_ref[...]


def _lane_bcast(v, k):
    idx = jnp.full((16, 1), k, dtype=jnp.int32)
    dn = lax.GatherDimensionNumbers(offset_dims=(), collapsed_slice_dims=(0,),
                                    start_index_map=(0,))
    return lax.gather(v, idx, dn, (1,),
                      mode=lax.GatherScatterMode.PROMISE_IN_BOUNDS)


def _sc_body(nchunk, vtab, i0, i1, i2, i3, w0, w1, w2, w3, out_hbm,
             idx_v0, idx_v1, w_v0, w_v1, rows_v0, rows_v1, out_v0, out_v1,
             gs0, gs1, iws0, iws1, os0, os1):
    cid = lax.axis_index("c")
    sid = lax.axis_index("s")
    wid = sid * NC + cid
    base = wid * (nchunk * CH)
    ihbms = [i0, i1, i2, i3]
    whbms = [w0, w1, w2, w3]
    idxs = [idx_v0, idx_v1]
    wvs = [w_v0, w_v1]
    rows = [rows_v0, rows_v1]
    outs = [out_v0, out_v1]
    gss = [gs0, gs1]
    iws = [iws0, iws1]
    oss = [os0, os1]

    def iw_pairs(g, b):
        off = pl.multiple_of((base + g * CH) * 16, 128)
        ps = []
        for c in range(4):
            ps.append((ihbms[c].at[pl.ds(off, CS)],
                       idxs[b].at[pl.ds(c * CS, CS)]))
            ps.append((whbms[c].at[pl.ds(off, CS)],
                       wvs[b].at[pl.ds(c * CS, CS)]))
        return ps

    def fire_iw(g, b):
        for src, dst in iw_pairs(g, b):
            pltpu.async_copy(src, dst, iws[b])

    def wait_iw(g, b):
        for src, dst in iw_pairs(g, b):
            pltpu.make_async_copy(src, dst, iws[b]).wait()

    def g_pairs(b):
        ps = []
        for j in range(4 * CS // 128):
            ps.append((vtab.at[idxs[b].at[pl.ds(j * 128, 128)]],
                       rows[b].at[pl.ds(j * 128, 128)]))
        return ps

    def fire_g(b):
        for src, dst in g_pairs(b):
            pltpu.async_copy(src, dst, gss[b])

    def wait_g(b):
        for src, dst in g_pairs(b):
            pltpu.make_async_copy(src, dst, gss[b]).wait()

    def out_slice(g):
        ib = pl.multiple_of(base + g * CH, CH)
        return out_hbm.at[pl.ds(ib * HD, CH * HD)]

    def compute(g, b):
        @pl.when(g >= 2)
        def _():
            pltpu.make_async_copy(outs[b], out_slice(g - 2), oss[b]).wait()

        w_v = wvs[b]
        rows_v = rows[b]
        out_v = outs[b]

        @plsc.parallel_loop(0, CH, 1, unroll=4)
        def per_q(qi):
            a0 = [jnp.zeros((16,), jnp.float32) for _ in range(4)]
            a1 = [jnp.zeros((16,), jnp.float32) for _ in range(4)]
            for c in range(4):
                w16 = w_v[pl.ds(c * CS + qi * 16, 16)]
                for k in range(16):
                    wb = _lane_bcast(w16, k)
                    r = c * CS + qi * 16 + k
                    lo, hi = plsc.unpack(
                        plsc.bitcast(rows_v[r, pl.ds(0, 16)], jnp.bfloat16),
                        format=plsc.PackFormat.INTERLEAVED)
                    a0[c] = a0[c] + wb * lo
                    a1[c] = a1[c] + wb * hi
            out_v[pl.ds(qi * HD, 16)] = (a0[0] + a0[1]) + (a0[2] + a0[3])
            out_v[pl.ds(qi * HD + 16, 16)] = (a1[0] + a1[1]) + (a1[2] + a1[3])
        pltpu.async_copy(out_v, out_slice(g), oss[b])

    fire_iw(0, 0)
    wait_iw(0, 0)
    fire_g(0)
    fire_iw(1, 1)

    def pair(p, _):
        for b in (0, 1):
            g = 2 * p + b

            @pl.when(g + 1 < nchunk)
            def _():
                wait_iw(g + 1, 1 - b)
                fire_g(1 - b)

            wait_g(b)
            compute(g, b)

            @pl.when(g + 2 < nchunk)
            def _():
                fire_iw(g + 2, b)
        return 0

    lax.fori_loop(0, nchunk // 2, pair, 0)
    pltpu.make_async_copy(outs[0], out_slice(nchunk - 2), oss[0]).wait()
    pltpu.make_async_copy(outs[1], out_slice(nchunk - 1), oss[1]).wait()


def kernel(query, reference_points, feat0, feat1, feat2, feat3,
           W_off, b_off, W_attn, b_attn, W_val, b_val, W_out, b_out):
    N, Q, D = query.shape
    f32 = jnp.float32
    feats = [feat0, feat1, feat2, feat3]
    featc = jnp.concatenate(
        [f.reshape(N, D, -1).transpose(0, 2, 1) for f in feats], axis=1)

    vtab = pl.pallas_call(
        _valproj_body,
        grid=(N, PIX // BP),
        in_specs=[
            pl.BlockSpec((1, BP, D), lambda n, p: (n, p, 0)),
            pl.BlockSpec((D, 128), lambda n, p: (0, 0)),
            pl.BlockSpec((D, 128), lambda n, p: (0, 0)),
            pl.BlockSpec((1, 128), lambda n, p: (0, 0)),
            pl.BlockSpec((1, 128), lambda n, p: (0, 0)),
        ],
        out_specs=pl.BlockSpec((BP, 128), lambda n, p: (n * (PIX // BP) + p, 0)),
        out_shape=jax.ShapeDtypeStruct((N * PIX, 128), jnp.uint32),
    )(featc, W_val.T[:, PERM_LO], W_val.T[:, PERM_HI],
      b_val[PERM_LO].reshape(1, 128), b_val[PERM_HI].reshape(1, 128))
    vtab_rows = vtab.reshape(N * PIX * NH, 16)

    rpx = jnp.broadcast_to(reference_points[:, :, 0:1], (N, Q, 128))
    rpy = jnp.broadcast_to(reference_points[:, :, 1:2], (N, Q, 128))
    wox = W_off[:, PERM_X]
    woy = W_off[:, PERM_Y]
    box = b_off[PERM_X].reshape(1, 128)
    boy = b_off[PERM_Y].reshape(1, 128)
    bat = b_attn.reshape(1, 128)

    qspec = pl.BlockSpec((1, BQ, 128), lambda n, qb: (n, qb, 0))
    wspec = pl.BlockSpec((D, 128), lambda n, qb: (0, 0))
    bspec = pl.BlockSpec((1, 128), lambda n, qb: (0, 0))
    outs = pl.pallas_call(
        _samp_body,
        grid=(N, Q // BQ),
        in_specs=[
            pl.BlockSpec((1, BQ, D), lambda n, qb: (n, qb, 0)),
            qspec, qspec, wspec, wspec, bspec, bspec, wspec, bspec,
            pl.BlockSpec((128, 128), lambda n, qb: (0, 0)),
            bspec, bspec, bspec, bspec,
        ],
        out_specs=[qspec] * 8,
        out_shape=[jax.ShapeDtypeStruct((N, Q, 128), jnp.int32)] * 4
                  + [jax.ShapeDtypeStruct((N, Q, 128), f32)] * 4,
    )(query, rpx, rpy, wox, woy, box, boy, W_attn, bat,
      jnp.asarray(GSUM), jnp.asarray(COL_W), jnp.asarray(COL_H),
      jnp.asarray(COL_WI), jnp.asarray(COL_BASE))
    idxs, ws = outs[:4], outs[4:]
    iflat = [i.reshape(-1) for i in idxs]
    wflat = [w.reshape(-1) for w in ws]

    TOT = N * Q * NH
    nchunk = TOT // (NW * CH)
    mesh = plsc.VectorSubcoreMesh(core_axis_name="c", subcore_axis_name="s")
    out_rows = pl.kernel(
        functools.partial(_sc_body, nchunk),
        out_type=jax.ShapeDtypeStruct((TOT * HD,), f32),
        mesh=mesh,
        scratch_types=[
            pltpu.VMEM((4 * CS,), jnp.int32),
            pltpu.VMEM((4 * CS,), jnp.int32),
            pltpu.VMEM((4 * CS,), f32),
            pltpu.VMEM((4 * CS,), f32),
            pltpu.VMEM((4 * CS, 16), jnp.uint32),
            pltpu.VMEM((4 * CS, 16), jnp.uint32),
            pltpu.VMEM((CH * HD,), f32),
            pltpu.VMEM((CH * HD,), f32),
            pltpu.SemaphoreType.DMA,
            pltpu.SemaphoreType.DMA,
            pltpu.SemaphoreType.DMA,
            pltpu.SemaphoreType.DMA,
            pltpu.SemaphoreType.DMA,
            pltpu.SemaphoreType.DMA,
        ],
        compiler_params=pltpu.CompilerParams(use_tc_tiling_on_sc=False,
                                             needs_layout_passes=False),
    )(vtab_rows, *iflat, *wflat)

    attn = out_rows.reshape(N * Q, D)
    final = pl.pallas_call(
        _out_body,
        grid=((N * Q) // BQ,),
        in_specs=[
            pl.BlockSpec((BQ, D), lambda i: (i, 0)),
            pl.BlockSpec((D, D), lambda i: (0, 0)),
            pl.BlockSpec((1, D), lambda i: (0, 0)),
        ],
        out_specs=pl.BlockSpec((BQ, D), lambda i: (i, 0)),
        out_shape=jax.ShapeDtypeStruct((N * Q, D), f32),
    )(attn, W_out, b_out.reshape(1, D))
    return final.reshape(N, Q, D)


# speedup vs baseline: 1.0275x; 1.0275x over previous
import functools
import numpy as np
import jax
import jax.numpy as jnp
from jax import lax
from jax.experimental import pallas as pl
from jax.experimental.pallas import tpu as pltpu
from jax.experimental.pallas import tpu_sc as plsc

NH, NL, NP_, HD = 8, 4, 4, 32
LVL_HW = [(64, 64), (32, 32), (16, 16), (8, 8)]
LVL_SIZES = [h * w for h, w in LVL_HW]
PIX = sum(LVL_SIZES)
LVL_BASE = np.concatenate([[0], np.cumsum(LVL_SIZES)[:-1]])

_l_of_col = (np.arange(128) % 16) // 4
_h_of_col = np.arange(128) // 16
COL_W = np.array([LVL_HW[l][1] for l in _l_of_col], np.float32).reshape(1, 128)
COL_H = np.array([LVL_HW[l][0] for l in _l_of_col], np.float32).reshape(1, 128)
COL_WI = (COL_W.astype(np.int64) * NH).astype(np.int32)
COL_BASE = (np.array([LVL_BASE[l] for l in _l_of_col], np.int64) * NH
            + _h_of_col).astype(np.int32).reshape(1, 128)
PERM_X = np.array([h * 32 + l * 8 + p * 2
                   for h in range(8) for l in range(4) for p in range(4)])
PERM_Y = PERM_X + 1
GSUM = np.kron(np.eye(8, dtype=np.float32), np.ones((16, 16), np.float32))
PERM_LO = np.array([h * 32 + k for h in range(8) for k in range(16)])
PERM_HI = PERM_LO + 16

BP = 544
BQ = 512
CH = 32
CS = CH * 16
NC, NS = 2, 16
NW = NC * NS


def _valproj_body(x_ref, wlo_ref, whi_ref, blo_ref, bhi_ref, o_ref):
    x = x_ref[0]
    ylo = jnp.dot(x, wlo_ref[...], preferred_element_type=jnp.float32) + blo_ref[...]
    yhi = jnp.dot(x, whi_ref[...], preferred_element_type=jnp.float32) + bhi_ref[...]
    lo = lax.bitcast_convert_type(ylo.astype(jnp.bfloat16), jnp.uint16).astype(jnp.uint32)
    hi = lax.bitcast_convert_type(yhi.astype(jnp.bfloat16), jnp.uint16).astype(jnp.uint32)
    o_ref[...] = lo | (hi << 16)


def _samp_body(q_ref, rpx_ref, rpy_ref, wox_ref, woy_ref, box_ref, boy_ref,
               wat_ref, bat_ref, g_ref, colw_ref, colh_ref, colwi_ref,
               colb_ref,
               i0_ref, i1_ref, i2_ref, i3_ref, w0_ref, w1_ref, w2_ref, w3_ref):
    q = q_ref[0]
    sox = jnp.dot(q, wox_ref[...], preferred_element_type=jnp.float32) + box_ref[...]
    soy = jnp.dot(q, woy_ref[...], preferred_element_type=jnp.float32) + boy_ref[...]
    aw = jnp.dot(q, wat_ref[...], preferred_element_type=jnp.float32) + bat_ref[...]
    m = jnp.max(aw, axis=-1, keepdims=True)
    e = jnp.exp(aw - m)
    s = jnp.dot(e, g_ref[...], preferred_element_type=jnp.float32)
    awf = e / s
    locx = rpx_ref[0] + sox
    locy = rpy_ref[0] + soy
    wv = colw_ref[...]
    hv = colh_ref[...]
    ix = locx * wv - 0.5
    iy = locy * hv - 0.5
    ix0 = jnp.floor(ix)
    iy0 = jnp.floor(iy)
    wx1 = ix - ix0
    wx0 = 1.0 - wx1
    wy1 = iy - iy0
    wy0 = 1.0 - wy1
    n = pl.program_id(0)
    nbase = n * (NH * PIX)
    wvi = colwi_ref[...]
    base = colb_ref[...]
    irefs = [i0_ref, i1_ref, i2_ref, i3_ref]
    wrefs = [w0_ref, w1_ref, w2_ref, w3_ref]
    for c, (dy, dx) in enumerate([(0, 0), (0, 1), (1, 0), (1, 1)]):
        fx = ix0 + dx
        fy = iy0 + dy
        valid = (fx >= 0) & (fx <= wv - 1) & (fy >= 0) & (fy <= hv - 1)
        ixc = jnp.clip(fx, 0.0, wv - 1).astype(jnp.int32)
        iyc = jnp.clip(fy, 0.0, hv - 1).astype(jnp.int32)
        row = nbase + base + iyc * wvi + ixc * NH
        wgt = jnp.where(valid, (wx1 if dx else wx0) * (wy1 if dy else wy0), 0.0) * awf
        irefs[c][0] = row
        wrefs[c][0] = wgt


def _out_body(x_ref, w_ref, b_ref, o_ref):
    o_ref[...] = jnp.dot(x_ref[...], w_ref[...],
                         preferred_element_type=jnp.float32) + b_ref[...]


def _lane_bcast(v, k):
    idx = jnp.full((16, 1), k, dtype=jnp.int32)
    dn = lax.GatherDimensionNumbers(offset_dims=(), collapsed_slice_dims=(0,),
                                    start_index_map=(0,))
    return lax.gather(v, idx, dn, (1,),
                      mode=lax.GatherScatterMode.PROMISE_IN_BOUNDS)


def _sc_body(nchunk, vtab, i0, i1, i2, i3, w0, w1, w2, w3, out_hbm,
             idx_v0, idx_v1, w_v0, w_v1, rows_v0, rows_v1, out_v0, out_v1,
             gs0, gs1, iws0, iws1, os0, os1):
    cid = lax.axis_index("c")
    sid = lax.axis_index("s")
    wid = sid * NC + cid
    base = wid * (nchunk * CH)
    ihbms = [i0, i1, i2, i3]
    whbms = [w0, w1, w2, w3]
    idxs = [idx_v0, idx_v1]
    wvs = [w_v0, w_v1]
    rows = [rows_v0, rows_v1]
    outs = [out_v0, out_v1]
    gss = [gs0, gs1]
    iws = [iws0, iws1]
    oss = [os0, os1]

    def iw_pairs(g, b):
        off = pl.multiple_of((base + g * CH) * 16, 128)
        ps = []
        for c in range(4):
            ps.append((ihbms[c].at[pl.ds(off, CS)],
                       idxs[b].at[pl.ds(c * CS, CS)]))
            ps.append((whbms[c].at[pl.ds(off, CS)],
                       wvs[b].at[pl.ds(c * CS, CS)]))
        return ps

    def fire_iw(g, b):
        for src, dst in iw_pairs(g, b):
            pltpu.async_copy(src, dst, iws[b])

    def wait_iw(g, b):
        for src, dst in iw_pairs(g, b):
            pltpu.make_async_copy(src, dst, iws[b]).wait()

    def g_pairs(b):
        ps = []
        for j in range(4 * CS // 128):
            ps.append((vtab.at[idxs[b].at[pl.ds(j * 128, 128)]],
                       rows[b].at[pl.ds(j * 128, 128)]))
        return ps

    def fire_g(b):
        for src, dst in g_pairs(b):
            pltpu.async_copy(src, dst, gss[b])

    def wait_g(b):
        for src, dst in g_pairs(b):
            pltpu.make_async_copy(src, dst, gss[b]).wait()

    def out_slice(g):
        ib = pl.multiple_of(base + g * CH, CH)
        return out_hbm.at[pl.ds(ib * HD, CH * HD)]

    def compute(g, b):
        @pl.when(g >= 2)
        def _():
            pltpu.make_async_copy(outs[b], out_slice(g - 2), oss[b]).wait()

        w_v = wvs[b]
        rows_v = rows[b]
        out_v = outs[b]

        @plsc.parallel_loop(0, CH, 1, unroll=2)
        def per_q(qi):
            a0 = [jnp.zeros((16,), jnp.float32) for _ in range(4)]
            a1 = [jnp.zeros((16,), jnp.float32) for _ in range(4)]
            for c in range(4):
                w16 = w_v[pl.ds(c * CS + qi * 16, 16)]
                for k in range(16):
                    wb = _lane_bcast(w16, k)
                    r = c * CS + qi * 16 + k
                    lo, hi = plsc.unpack(
                        plsc.bitcast(rows_v[r, pl.ds(0, 16)], jnp.bfloat16),
                        format=plsc.PackFormat.INTERLEAVED)
                    a0[c] = a0[c] + wb * lo
                    a1[c] = a1[c] + wb * hi
            out_v[pl.ds(qi * HD, 16)] = (a0[0] + a0[1]) + (a0[2] + a0[3])
            out_v[pl.ds(qi * HD + 16, 16)] = (a1[0] + a1[1]) + (a1[2] + a1[3])
        pltpu.async_copy(out_v, out_slice(g), oss[b])

    fire_iw(0, 0)
    wait_iw(0, 0)
    fire_g(0)
    fire_iw(1, 1)

    def pair(p, _):
        for b in (0, 1):
            g = 2 * p + b

            @pl.when(g + 1 < nchunk)
            def _():
                wait_iw(g + 1, 1 - b)
                fire_g(1 - b)

            wait_g(b)
            compute(g, b)

            @pl.when(g + 2 < nchunk)
            def _():
                fire_iw(g + 2, b)
        return 0

    lax.fori_loop(0, nchunk // 2, pair, 0)
    pltpu.make_async_copy(outs[0], out_slice(nchunk - 2), oss[0]).wait()
    pltpu.make_async_copy(outs[1], out_slice(nchunk - 1), oss[1]).wait()


def kernel(query, reference_points, feat0, feat1, feat2, feat3,
           W_off, b_off, W_attn, b_attn, W_val, b_val, W_out, b_out):
    N, Q, D = query.shape
    f32 = jnp.float32
    feats = [feat0, feat1, feat2, feat3]
    featc = jnp.concatenate(
        [f.reshape(N, D, -1).transpose(0, 2, 1) for f in feats], axis=1)

    vtab = pl.pallas_call(
        _valproj_body,
        grid=(N, PIX // BP),
        in_specs=[
            pl.BlockSpec((1, BP, D), lambda n, p: (n, p, 0)),
            pl.BlockSpec((D, 128), lambda n, p: (0, 0)),
            pl.BlockSpec((D, 128), lambda n, p: (0, 0)),
            pl.BlockSpec((1, 128), lambda n, p: (0, 0)),
            pl.BlockSpec((1, 128), lambda n, p: (0, 0)),
        ],
        out_specs=pl.BlockSpec((BP, 128), lambda n, p: (n * (PIX // BP) + p, 0)),
        out_shape=jax.ShapeDtypeStruct((N * PIX, 128), jnp.uint32),
    )(featc, W_val.T[:, PERM_LO], W_val.T[:, PERM_HI],
      b_val[PERM_LO].reshape(1, 128), b_val[PERM_HI].reshape(1, 128))
    vtab_rows = vtab.reshape(N * PIX * NH, 16)

    rpx = jnp.broadcast_to(reference_points[:, :, 0:1], (N, Q, 128))
    rpy = jnp.broadcast_to(reference_points[:, :, 1:2], (N, Q, 128))
    wox = W_off[:, PERM_X]
    woy = W_off[:, PERM_Y]
    box = b_off[PERM_X].reshape(1, 128)
    boy = b_off[PERM_Y].reshape(1, 128)
    bat = b_attn.reshape(1, 128)

    qspec = pl.BlockSpec((1, BQ, 128), lambda n, qb: (n, qb, 0))
    wspec = pl.BlockSpec((D, 128), lambda n, qb: (0, 0))
    bspec = pl.BlockSpec((1, 128), lambda n, qb: (0, 0))
    outs = pl.pallas_call(
        _samp_body,
        grid=(N, Q // BQ),
        in_specs=[
            pl.BlockSpec((1, BQ, D), lambda n, qb: (n, qb, 0)),
            qspec, qspec, wspec, wspec, bspec, bspec, wspec, bspec,
            pl.BlockSpec((128, 128), lambda n, qb: (0, 0)),
            bspec, bspec, bspec, bspec,
        ],
        out_specs=[qspec] * 8,
        out_shape=[jax.ShapeDtypeStruct((N, Q, 128), jnp.int32)] * 4
                  + [jax.ShapeDtypeStruct((N, Q, 128), f32)] * 4,
    )(query, rpx, rpy, wox, woy, box, boy, W_attn, bat,
      jnp.asarray(GSUM), jnp.asarray(COL_W), jnp.asarray(COL_H),
      jnp.asarray(COL_WI), jnp.asarray(COL_BASE))
    idxs, ws = outs[:4], outs[4:]
    iflat = [i.reshape(-1) for i in idxs]
    wflat = [w.reshape(-1) for w in ws]

    TOT = N * Q * NH
    nchunk = TOT // (NW * CH)
    mesh = plsc.VectorSubcoreMesh(core_axis_name="c", subcore_axis_name="s")
    out_rows = pl.kernel(
        functools.partial(_sc_body, nchunk),
        out_type=jax.ShapeDtypeStruct((TOT * HD,), f32),
        mesh=mesh,
        scratch_types=[
            pltpu.VMEM((4 * CS,), jnp.int32),
            pltpu.VMEM((4 * CS,), jnp.int32),
            pltpu.VMEM((4 * CS,), f32),
            pltpu.VMEM((4 * CS,), f32),
            pltpu.VMEM((4 * CS, 16), jnp.uint32),
            pltpu.VMEM((4 * CS, 16), jnp.uint32),
            pltpu.VMEM((CH * HD,), f32),
            pltpu.VMEM((CH * HD,), f32),
            pltpu.SemaphoreType.DMA,
            pltpu.SemaphoreType.DMA,
            pltpu.SemaphoreType.DMA,
            pltpu.SemaphoreType.DMA,
            pltpu.SemaphoreType.DMA,
            pltpu.SemaphoreType.DMA,
        ],
        compiler_params=pltpu.CompilerParams(use_tc_tiling_on_sc=False,
                                             needs_layout_passes=False),
    )(vtab_rows, *iflat, *wflat)

    attn = out_rows.reshape(N * Q, D)
    final = pl.pallas_call(
        _out_body,
        grid=((N * Q) // BQ,),
        in_specs=[
            pl.BlockSpec((BQ, D), lambda i: (i, 0)),
            pl.BlockSpec((D, D), lambda i: (0, 0)),
            pl.BlockSpec((1, D), lambda i: (0, 0)),
        ],
        out_specs=pl.BlockSpec((BQ, D), lambda i: (i, 0)),
        out_shape=jax.ShapeDtypeStruct((N * Q, D), f32),
    )(attn, W_out, b_out.reshape(1, D))
    return final.reshape(N, Q, D)
